# baseline (speedup 1.0000x reference)
import functools

import jax
import jax.numpy as jnp
from jax import lax
from jax.experimental import pallas as pl
from jax.experimental.pallas import tpu as pltpu
from jax.experimental.pallas import tpu_sc as plsc

N = 100000
E = 1600000

NT = 100352
TC_BLK = 2048
TC_GRID = NT // TC_BLK
TP16 = NT * 16 // 128
TP32 = NT * 32 // 128
PB16 = TC_BLK * 16 // 128
PB32 = TC_BLK * 32 // 128

NSC = 2
NTILE = 16
ROWS_PER_TILE = NT // NTILE
CHUNK = 128
K = 6
NCH = E // CHUNK
BULK = 12480
TAIL = NCH - BULK
C_T1 = BULK // NTILE
G1 = C_T1 // K
C_T2 = BULK // (NSC * NTILE)
G2 = C_T2 // K

_mesh = plsc.VectorSubcoreMesh(core_axis_name="c", subcore_axis_name="s")

_SC_SCRATCH = [
    pltpu.VMEM((2, K, CHUNK), jnp.int32),
    pltpu.VMEM((2, K, CHUNK), jnp.int32),
    pltpu.VMEM((2, K, CHUNK, 16), jnp.float32),
    pltpu.VMEM((CHUNK, 16), jnp.float32),
    pltpu.VMEM_SHARED((NT, 16), jnp.float32),
    pltpu.SemaphoreType.DMA,
    pltpu.SemaphoreType.DMA,
]

_SC_OUT = [jax.ShapeDtypeStruct((NT, 16), jnp.float32),
           jax.ShapeDtypeStruct((NT, 16), jnp.float32)]


def _sc_zero_acc(acc, zbuf, s):
    def zb(i, carry):
        zbuf[i, :] = jnp.zeros((16,), jnp.float32)
        return carry
    lax.fori_loop(0, CHUNK, zb, 0)
    base = s * ROWS_PER_TILE

    def zacc(i, carry):
        pltpu.sync_copy(zbuf, acc.at[pl.ds(base + i * CHUNK, CHUNK)])
        return carry
    lax.fori_loop(0, ROWS_PER_TILE // CHUNK, zacc, 0)


def _sc_writeout(acc, out_a, out_b, c, s):
    sl = pl.ds(s * ROWS_PER_TILE, ROWS_PER_TILE)

    @pl.when(c == 0)
    def _():
        pltpu.sync_copy(acc.at[sl], out_a.at[sl])

    @pl.when(c != 0)
    def _():
        pltpu.sync_copy(acc.at[sl], out_b.at[sl])


def _sc_segment_sum(fire_gathers, gather_one, drain_tab, e3, acc,
                    sidx, didx, rows, gsem, ssem,
                    chunk0, ngroups, tail0, ntail, tail_stride):
    def load_idx(g, b):
        ch0 = chunk0 + g * K
        pltpu.sync_copy(e3.at[pl.ds(ch0, K)], sidx.at[b])
        pltpu.sync_copy(e3.at[pl.ds(NCH + ch0, K)], didx.at[b])

    def drain(b, sem):
        for j in range(K):
            pltpu.make_async_copy(
                drain_tab.at[pl.ds(0, CHUNK)], rows.at[b, j], sem).wait()

    def fire_scatters(b):
        for j in range(K):
            pltpu.async_copy(rows.at[b, j], acc.at[didx.at[b, j]], ssem,
                             add=True)

    load_idx(0, 0)
    fire_gathers(0)

    def grp(g, carry):
        b = lax.rem(g, 2)
        nb = lax.rem(g + 1, 2)

        @pl.when(g + 1 < ngroups)
        def _():
            load_idx(g + 1, nb)
        drain(b, gsem)
        fire_scatters(b)

        @pl.when(g + 1 < ngroups)
        def _():
            fire_gathers(nb)
        drain(b, ssem)
        return carry
    lax.fori_loop(0, ngroups, grp, 0)

    def tl(k, carry):
        ch = tail0 + tail_stride * k
        pltpu.sync_copy(e3.at[pl.ds(ch, 1)], sidx.at[0, pl.ds(0, 1)])
        pltpu.sync_copy(e3.at[pl.ds(NCH + ch, 1)], didx.at[0, pl.ds(0, 1)])
        gather_one()
        pltpu.sync_copy(rows.at[0, 0], acc.at[didx.at[0, 0]], add=True)
        return carry
    lax.fori_loop(0, ntail, tl, 0)


@functools.partial(
    pl.kernel,
    out_type=_SC_OUT,
    mesh=_mesh,
    scratch_types=_SC_SCRATCH,
    compiler_params=pltpu.CompilerParams(use_tc_tiling_on_sc=False),
)
def _sc_layer1(tab_a, tab_b, e3, out_a, out_b,
               sidx, didx, rows, zbuf, acc, gsem, ssem):
    c = lax.axis_index("c")
    s = lax.axis_index("s")
    _sc_zero_acc(acc, zbuf, s)
    plsc.subcore_barrier()

    def fire_gathers(b):
        for j in range(K):
            @pl.when(c == 0)
            def _():
                pltpu.async_copy(tab_a.at[sidx.at[b, j]], rows.at[b, j], gsem)

            @pl.when(c != 0)
            def _():
                pltpu.async_copy(tab_b.at[sidx.at[b, j]], rows.at[b, j], gsem)

    def gather_one():
        @pl.when(c == 0)
        def _():
            pltpu.async_copy(tab_a.at[sidx.at[0, 0]], rows.at[0, 0],
                             gsem).wait()

        @pl.when(c != 0)
        def _():
            pltpu.async_copy(tab_b.at[sidx.at[0, 0]], rows.at[0, 0],
                             gsem).wait()

    ntail = jnp.where(s < TAIL - NTILE, 2, 1)
    _sc_segment_sum(fire_gathers, gather_one, tab_a, e3, acc,
                    sidx, didx, rows, gsem, ssem,
                    chunk0=s * C_T1, ngroups=G1,
                    tail0=BULK + s, ntail=ntail, tail_stride=NTILE)

    plsc.subcore_barrier()
    _sc_writeout(acc, out_a, out_b, c, s)


@functools.partial(
    pl.kernel,
    out_type=_SC_OUT,
    mesh=_mesh,
    scratch_types=_SC_SCRATCH,
    compiler_params=pltpu.CompilerParams(use_tc_tiling_on_sc=False),
)
def _sc_layer2(tab, e3, out_a, out_b,
               sidx, didx, rows, zbuf, acc, gsem, ssem):
    c = lax.axis_index("c")
    s = lax.axis_index("s")
    wid = c * NTILE + s
    _sc_zero_acc(acc, zbuf, s)
    plsc.subcore_barrier()

    def fire_gathers(b):
        for j in range(K):
            pltpu.async_copy(tab.at[sidx.at[b, j]], rows.at[b, j], gsem)

    def gather_one():
        pltpu.async_copy(tab.at[sidx.at[0, 0]], rows.at[0, 0], gsem).wait()

    ntail = jnp.where(wid < TAIL, 1, 0)
    _sc_segment_sum(fire_gathers, gather_one, tab, e3, acc,
                    sidx, didx, rows, gsem, ssem,
                    chunk0=wid * C_T2, ngroups=G2,
                    tail0=BULK + wid, ntail=ntail,
                    tail_stride=NSC * NTILE)

    plsc.subcore_barrier()
    _sc_writeout(acc, out_a, out_b, c, s)


def _tc_a_body(xr_ref, sel_ref, wa_ref, wb_ref, v_ref,
               ha_ref, hb_ref, ra_ref, rb_ref):
    xr = xr_ref[...]
    xa = jnp.dot(xr, sel_ref[0], preferred_element_type=jnp.float32)
    xb = jnp.dot(xr, sel_ref[1], preferred_element_type=jnp.float32)
    outs = [ha_ref, hb_ref, ra_ref, rb_ref]
    for i in range(4):
        outs[i][...] = (
            jnp.dot(xa, wa_ref[i], preferred_element_type=jnp.float32)
            + jnp.dot(xb, wb_ref[i], preferred_element_type=jnp.float32)
            + v_ref[i:i + 1, :])


def _tc_b_body(a_ref, b_ref, ra_ref, rb_ref, s5_ref, w2_ref, b2_ref,
               h2_ref, r2_ref):
    a = a_ref[...]
    b = b_ref[...]
    recip = 1.0 / jnp.maximum(
        jnp.dot(b, s5_ref[...], preferred_element_type=jnp.float32), 1.0)
    y1a = jnp.maximum(a * recip + ra_ref[...], 0.0)
    y1b = jnp.maximum(b * recip + rb_ref[...], 0.0)
    h2_ref[...] = (
        jnp.dot(y1a, w2_ref[0], preferred_element_type=jnp.float32)
        + jnp.dot(y1b, w2_ref[1], preferred_element_type=jnp.float32))
    r2_ref[...] = (
        jnp.dot(y1a, w2_ref[2], preferred_element_type=jnp.float32)
        + jnp.dot(y1b, w2_ref[3], preferred_element_type=jnp.float32)
        + b2_ref[...])


def _tc_c_body(a_ref, b_ref, r2_ref, cb_ref, s5_ref, out_ref):
    recip = 1.0 / jnp.maximum(
        jnp.dot(cb_ref[...], s5_ref[...], preferred_element_type=jnp.float32),
        1.0)
    out_ref[...] = jnp.maximum(
        (a_ref[...] + b_ref[...]) * recip + r2_ref[...], 0.0)


def _pk_spec():
    return pl.BlockSpec((PB16, 128), lambda i: (i, 0))


def _full_spec(*dims):
    nd = len(dims)
    return pl.BlockSpec(dims, lambda i: (0,) * nd)


def _pad16(m, rows=16):
    return jnp.zeros((rows, 16), jnp.float32).at[:m.shape[0], :m.shape[1]].set(m)


def kernel(x, edge_index, W_l1, W_r1, b1, W_l2, W_r2, b2):
    f32 = jnp.float32
    t16 = jax.ShapeDtypeStruct((TP16, 128), f32)
    eye8 = jnp.eye(8, dtype=f32)

    def bd(m):
        return jnp.kron(eye8, _pad16(m))

    e3 = edge_index.reshape(2 * NCH, CHUNK)

    xr = x.reshape(N // 8, 256)
    sel = jnp.stack([
        jnp.kron(eye8, jnp.zeros((32, 16), f32).at[:16, :].set(jnp.eye(16))),
        jnp.kron(eye8, jnp.zeros((32, 16), f32).at[16:, :].set(jnp.eye(16)))])

    wa = jnp.stack([bd(W_l1[:16, :16]), bd(W_l1[:16, 16:21]),
                    bd(W_r1[:16, :16]), bd(W_r1[:16, 16:21])])
    wb = jnp.stack([bd(W_l1[16:, :16]), bd(W_l1[16:, 16:21]),
                    bd(W_r1[16:, :16]), bd(W_r1[16:, 16:21])])
    c5 = jnp.tile(jnp.zeros((16,), f32).at[5].set(1.0), 8)
    v = jnp.stack([jnp.zeros((128,), f32), c5,
                   jnp.tile(b1[:16], 8),
                   jnp.tile(jnp.zeros((16,), f32).at[:5].set(b1[16:]), 8)])
    s5 = jnp.kron(eye8, jnp.zeros((16, 16), f32).at[5, :].set(1.0))
    w2 = jnp.stack([bd(W_l2[:16, :]), bd(W_l2[16:, :]),
                    bd(W_r2[:16, :]), bd(W_r2[16:, :])])
    b2row = jnp.tile(jnp.zeros((16,), f32).at[:10].set(b2), 8).reshape(1, 128)

    h1a, h1b, r1a, r1b = pl.pallas_call(
        _tc_a_body,
        grid=(TC_GRID,),
        in_specs=[pl.BlockSpec((PB16, 256), lambda i: (i, 0)),
                  _full_spec(2, 256, 128), _full_spec(4, 128, 128),
                  _full_spec(4, 128, 128), _full_spec(4, 128)],
        out_specs=[_pk_spec()] * 4,
        out_shape=[t16] * 4,
    )(xr, sel, wa, wb, v)

    agg1a, agg1b = _sc_layer1(h1a.reshape(NT, 16), h1b.reshape(NT, 16), e3)

    h2, r2 = pl.pallas_call(
        _tc_b_body,
        grid=(TC_GRID,),
        in_specs=[_pk_spec(), _pk_spec(), _pk_spec(), _pk_spec(),
                  _full_spec(128, 128), _full_spec(4, 128, 128),
                  _full_spec(1, 128)],
        out_specs=[_pk_spec(), _pk_spec()],
        out_shape=[t16, t16],
    )(agg1a.reshape(TP16, 128), agg1b.reshape(TP16, 128), r1a, r1b,
      s5, w2, b2row)

    agg2a, agg2b = _sc_layer2(h2.reshape(NT, 16), e3)

    outp = pl.pallas_call(
        _tc_c_body,
        grid=(TC_GRID,),
        in_specs=[_pk_spec(), _pk_spec(), _pk_spec(), _pk_spec(),
                  _full_spec(128, 128)],
        out_specs=_pk_spec(),
        out_shape=t16,
    )(agg2a.reshape(TP16, 128), agg2b.reshape(TP16, 128), r2,
      agg1b.reshape(TP16, 128), s5)

    return outp[:N * 16 // 128].reshape(N // 8, 8, 16)[:, :, :10].reshape(N, 10)


# speedup vs baseline: 1.0844x; 1.0844x over previous
import functools

import jax
import jax.numpy as jnp
from jax import lax
from jax.experimental import pallas as pl
from jax.experimental.pallas import tpu as pltpu
from jax.experimental.pallas import tpu_sc as plsc

N = 100000
E = 1600000

NT = 100352
TC_BLK = 2048
TC_GRID = NT // TC_BLK
TP16 = NT * 16 // 128
TP32 = NT * 32 // 128
PB16 = TC_BLK * 16 // 128
PB32 = TC_BLK * 32 // 128

NSC = 2
NTILE = 16
ROWS_PER_TILE = NT // NTILE
CHUNK = 128
K = 6
NCH = E // CHUNK
BULK = 12480
TAIL = NCH - BULK
C_T1 = BULK // NTILE
G1 = C_T1 // K
C_T2 = BULK // (NSC * NTILE)
G2 = C_T2 // K

_mesh = plsc.VectorSubcoreMesh(core_axis_name="c", subcore_axis_name="s")

_SC_SCRATCH = [
    pltpu.VMEM((2, K, CHUNK), jnp.int32),
    pltpu.VMEM((2, K, CHUNK), jnp.int32),
    pltpu.VMEM((2, K, CHUNK, 16), jnp.float32),
    pltpu.VMEM((CHUNK, 16), jnp.float32),
    pltpu.VMEM_SHARED((NT, 16), jnp.float32),
    pltpu.SemaphoreType.DMA,
    pltpu.SemaphoreType.DMA,
]

_SC_OUT = [jax.ShapeDtypeStruct((NT, 16), jnp.float32),
           jax.ShapeDtypeStruct((NT, 16), jnp.float32)]


def _sc_zero_acc(acc, zbuf, s):
    def zb(i, carry):
        zbuf[i, :] = jnp.zeros((16,), jnp.float32)
        return carry
    lax.fori_loop(0, CHUNK, zb, 0)
    base = s * ROWS_PER_TILE

    def zacc(i, carry):
        pltpu.sync_copy(zbuf, acc.at[pl.ds(base + i * CHUNK, CHUNK)])
        return carry
    lax.fori_loop(0, ROWS_PER_TILE // CHUNK, zacc, 0)


def _sc_writeout(acc, out_a, out_b, c, s):
    sl = pl.ds(s * ROWS_PER_TILE, ROWS_PER_TILE)

    @pl.when(c == 0)
    def _():
        pltpu.sync_copy(acc.at[sl], out_a.at[sl])

    @pl.when(c != 0)
    def _():
        pltpu.sync_copy(acc.at[sl], out_b.at[sl])


def _sc_segment_sum(fire_gathers, gather_one, drain_tab, e3, acc,
                    sidx, didx, rows, gsem, ssem,
                    chunk0, ngroups, tail0, ntail, tail_stride):
    def load_idx(g, b):
        ch0 = chunk0 + g * K
        pltpu.sync_copy(e3.at[pl.ds(ch0, K)], sidx.at[b])
        pltpu.sync_copy(e3.at[pl.ds(NCH + ch0, K)], didx.at[b])

    def drain(b, sem):
        for j in range(K):
            pltpu.make_async_copy(
                drain_tab.at[pl.ds(0, CHUNK)], rows.at[b, j], sem).wait()

    def fire_scatters(b):
        for j in range(K):
            pltpu.async_copy(rows.at[b, j], acc.at[didx.at[b, j]], ssem,
                             add=True)

    load_idx(0, 0)
    fire_gathers(0)

    def grp(g, carry):
        b = lax.rem(g, 2)
        nb = lax.rem(g + 1, 2)

        @pl.when(g + 1 < ngroups)
        def _():
            load_idx(g + 1, nb)
        drain(b, gsem)
        fire_scatters(b)

        @pl.when(g + 1 < ngroups)
        def _():
            fire_gathers(nb)
        drain(b, ssem)
        return carry
    lax.fori_loop(0, ngroups, grp, 0)

    def tl(k, carry):
        ch = tail0 + tail_stride * k
        pltpu.sync_copy(e3.at[pl.ds(ch, 1)], sidx.at[0, pl.ds(0, 1)])
        pltpu.sync_copy(e3.at[pl.ds(NCH + ch, 1)], didx.at[0, pl.ds(0, 1)])
        gather_one()
        pltpu.sync_copy(rows.at[0, 0], acc.at[didx.at[0, 0]], add=True)
        return carry
    lax.fori_loop(0, ntail, tl, 0)


@functools.partial(
    pl.kernel,
    out_type=_SC_OUT,
    mesh=_mesh,
    scratch_types=_SC_SCRATCH,
    compiler_params=pltpu.CompilerParams(use_tc_tiling_on_sc=False),
)
def _sc_layer1(tab_a, tab_b, e3, out_a, out_b,
               sidx, didx, rows, zbuf, acc, gsem, ssem):
    c = lax.axis_index("c")
    s = lax.axis_index("s")
    _sc_zero_acc(acc, zbuf, s)
    plsc.subcore_barrier()

    def fire_gathers(b):
        for j in range(K):
            @pl.when(c == 0)
            def _():
                pltpu.async_copy(tab_a.at[sidx.at[b, j]], rows.at[b, j], gsem)

            @pl.when(c != 0)
            def _():
                pltpu.async_copy(tab_b.at[sidx.at[b, j]], rows.at[b, j], gsem)

    def gather_one():
        @pl.when(c == 0)
        def _():
            pltpu.async_copy(tab_a.at[sidx.at[0, 0]], rows.at[0, 0],
                             gsem).wait()

        @pl.when(c != 0)
        def _():
            pltpu.async_copy(tab_b.at[sidx.at[0, 0]], rows.at[0, 0],
                             gsem).wait()

    ntail = jnp.where(s < TAIL - NTILE, 2, 1)
    _sc_segment_sum(fire_gathers, gather_one, tab_a, e3, acc,
                    sidx, didx, rows, gsem, ssem,
                    chunk0=s * C_T1, ngroups=G1,
                    tail0=BULK + s, ntail=ntail, tail_stride=NTILE)

    plsc.subcore_barrier()
    _sc_writeout(acc, out_a, out_b, c, s)


@functools.partial(
    pl.kernel,
    out_type=_SC_OUT,
    mesh=_mesh,
    scratch_types=_SC_SCRATCH,
    compiler_params=pltpu.CompilerParams(use_tc_tiling_on_sc=False),
)
def _sc_layer2(tab, e3, out_a, out_b,
               sidx, didx, rows, zbuf, acc, gsem, ssem):
    c = lax.axis_index("c")
    s = lax.axis_index("s")
    wid = c * NTILE + s
    _sc_zero_acc(acc, zbuf, s)
    plsc.subcore_barrier()

    def fire_gathers(b):
        for j in range(K):
            pltpu.async_copy(tab.at[sidx.at[b, j]], rows.at[b, j], gsem)

    def gather_one():
        pltpu.async_copy(tab.at[sidx.at[0, 0]], rows.at[0, 0], gsem).wait()

    ntail = jnp.where(wid < TAIL, 1, 0)
    _sc_segment_sum(fire_gathers, gather_one, tab, e3, acc,
                    sidx, didx, rows, gsem, ssem,
                    chunk0=wid * C_T2, ngroups=G2,
                    tail0=BULK + wid, ntail=ntail,
                    tail_stride=NSC * NTILE)

    plsc.subcore_barrier()
    _sc_writeout(acc, out_a, out_b, c, s)


def _tc_a_body(xr_ref, sel_ref, wa_ref, wb_ref, v_ref,
               ha_ref, hb_ref, ra_ref, rb_ref):
    xr = xr_ref[...]
    xa = jnp.dot(xr, sel_ref[0], preferred_element_type=jnp.float32)
    xb = jnp.dot(xr, sel_ref[1], preferred_element_type=jnp.float32)
    outs = [ha_ref, hb_ref, ra_ref, rb_ref]
    for i in range(4):
        outs[i][...] = (
            jnp.dot(xa, wa_ref[i], preferred_element_type=jnp.float32)
            + jnp.dot(xb, wb_ref[i], preferred_element_type=jnp.float32)
            + v_ref[i:i + 1, :])


def _tc_b_body(a_ref, b_ref, ra_ref, rb_ref, s5_ref, w2_ref, b2_ref,
               h2_ref, r2_ref):
    a = a_ref[...]
    b = b_ref[...]
    recip = 1.0 / jnp.maximum(
        jnp.dot(b, s5_ref[...], preferred_element_type=jnp.float32), 1.0)
    y1a = jnp.maximum(a * recip + ra_ref[...], 0.0)
    y1b = jnp.maximum(b * recip + rb_ref[...], 0.0)
    h2_ref[...] = (
        jnp.dot(y1a, w2_ref[0], preferred_element_type=jnp.float32)
        + jnp.dot(y1b, w2_ref[1], preferred_element_type=jnp.float32))
    r2_ref[...] = (
        jnp.dot(y1a, w2_ref[2], preferred_element_type=jnp.float32)
        + jnp.dot(y1b, w2_ref[3], preferred_element_type=jnp.float32)
        + b2_ref[...])


def _tc_c_body(a_ref, b_ref, r2_ref, cb_ref, s5_ref, out_ref):
    recip = 1.0 / jnp.maximum(
        jnp.dot(cb_ref[...], s5_ref[...], preferred_element_type=jnp.float32),
        1.0)
    out_ref[...] = jnp.maximum(
        (a_ref[...] + b_ref[...]) * recip + r2_ref[...], 0.0)


def _pk_spec():
    return pl.BlockSpec((PB16, 128), lambda i: (i, 0))


def _full_spec(*dims):
    nd = len(dims)
    return pl.BlockSpec(dims, lambda i: (0,) * nd)


def _pad16(m, rows=16):
    return jnp.zeros((rows, 16), jnp.float32).at[:m.shape[0], :m.shape[1]].set(m)


def kernel(x, edge_index, W_l1, W_r1, b1, W_l2, W_r2, b2):
    f32 = jnp.float32
    t16 = jax.ShapeDtypeStruct((TP16, 128), f32)
    eye8 = jnp.eye(8, dtype=f32)

    def bd(m):
        return jnp.kron(eye8, _pad16(m))

    e3 = edge_index.reshape(2 * NCH, CHUNK)

    xr = x.reshape(N // 8, 256)
    sel = jnp.stack([
        jnp.kron(eye8, jnp.zeros((32, 16), f32).at[:16, :].set(jnp.eye(16))),
        jnp.kron(eye8, jnp.zeros((32, 16), f32).at[16:, :].set(jnp.eye(16)))])

    wa = jnp.stack([bd(W_l1[:16, :16]), bd(W_l1[:16, 16:21]),
                    bd(W_r1[:16, :16]), bd(W_r1[:16, 16:21])])
    wb = jnp.stack([bd(W_l1[16:, :16]), bd(W_l1[16:, 16:21]),
                    bd(W_r1[16:, :16]), bd(W_r1[16:, 16:21])])
    c5 = jnp.tile(jnp.zeros((16,), f32).at[5].set(1.0), 8)
    v = jnp.stack([jnp.zeros((128,), f32), c5,
                   jnp.tile(b1[:16], 8),
                   jnp.tile(jnp.zeros((16,), f32).at[:5].set(b1[16:]), 8)])
    s5 = jnp.kron(eye8, jnp.zeros((16, 16), f32).at[5, :].set(1.0))
    w2 = jnp.stack([bd(W_l2[:16, :]), bd(W_l2[16:, :]),
                    bd(W_r2[:16, :]), bd(W_r2[16:, :])])
    b2row = jnp.tile(jnp.zeros((16,), f32).at[:10].set(b2), 8).reshape(1, 128)

    h1a, h1b, r1a, r1b = pl.pallas_call(
        _tc_a_body,
        grid=(TC_GRID,),
        in_specs=[pl.BlockSpec((PB16, 256), lambda i: (i, 0)),
                  _full_spec(2, 256, 128), _full_spec(4, 128, 128),
                  _full_spec(4, 128, 128), _full_spec(4, 128)],
        out_specs=[_pk_spec()] * 4,
        out_shape=[t16] * 4,
    )(xr, sel, wa, wb, v)

    agg1a, agg1b = _sc_layer1(h1a.reshape(NT, 16), h1b.reshape(NT, 16), e3)

    h2, r2 = pl.pallas_call(
        _tc_b_body,
        grid=(TC_GRID,),
        in_specs=[_pk_spec(), _pk_spec(), _pk_spec(), _pk_spec(),
                  _full_spec(128, 128), _full_spec(4, 128, 128),
                  _full_spec(1, 128)],
        out_specs=[_pk_spec(), _pk_spec()],
        out_shape=[t16, t16],
    )(agg1a.reshape(TP16, 128), agg1b.reshape(TP16, 128), r1a, r1b,
      s5, w2, b2row)

    agg2a, agg2b = _sc_layer2(h2.reshape(NT, 16), e3)

    outp = pl.pallas_call(
        _tc_c_body,
        grid=(TC_GRID,),
        in_specs=[_pk_spec(), _pk_spec(), _pk_spec(), _pk_spec(),
                  _full_spec(128, 128)],
        out_specs=_pk_spec(),
        out_shape=t16,
    )(agg2a.reshape(TP16, 128), agg2b.reshape(TP16, 128), r2,
      agg1b.reshape(TP16, 128), s5)

    return outp[:N * 16 // 128].reshape(N, 16)[:, :10]
